# baseline (speedup 1.0000x reference)
import functools

import jax
import jax.numpy as jnp
from jax import lax
from jax.experimental import pallas as pl
from jax.experimental.pallas import tpu as pltpu
from jax.experimental.pallas import tpu_sc as plsc

N = 10000
E = 320000
FH = 128
NC = 2
NS = 16
NW = NC * NS
CHUNK = 125
EDGES_PER_TILE = E // NW
CHUNKS_PER_TILE = EDGES_PER_TILE // CHUNK
N_PAD = 10240
ROWS_PER_TILE = N_PAD // NS
CH = 64
CPT = 160
E_PAD = NW * CPT * CH
NPHASE = 4
PHASE_CHUNKS = CPT // NPHASE
DEGW = 16

def _deg_body(src_hbm, ones_hbm, zeros_hbm, out_hbm, idx_s, ones_v, acc,
              d0, d1, d2, d3):
    cid = lax.axis_index("c")
    sid = lax.axis_index("s")
    wid = cid * NS + sid
    dsems = (d0, d1, d2, d3)
    pltpu.sync_copy(zeros_hbm, acc.at[pl.ds(sid * ROWS_PER_TILE, ROWS_PER_TILE)])
    pltpu.sync_copy(ones_hbm, ones_v)
    pltpu.sync_copy(src_hbm.at[pl.ds(wid * CHUNKS_PER_TILE, CHUNKS_PER_TILE)], idx_s)
    plsc.subcore_barrier()

    def body(g, carry):
        for k in range(4):
            pltpu.async_copy(ones_v, acc.at[idx_s.at[4 * g + k]], dsems[k], add=True)
        for k in range(4):
            pltpu.make_async_copy(ones_v, acc.at[idx_s.at[4 * g + k]], dsems[k]).wait()
        return carry

    lax.fori_loop(0, CHUNKS_PER_TILE // 4, body, 0)
    plsc.subcore_barrier()
    pltpu.sync_copy(acc.at[pl.ds(sid * ROWS_PER_TILE, ROWS_PER_TILE)],
                    out_hbm.at[cid, pl.ds(sid * ROWS_PER_TILE, ROWS_PER_TILE)])


@functools.cache
def _get_deg_kernel():
    mesh = plsc.VectorSubcoreMesh(core_axis_name="c", subcore_axis_name="s")
    return pl.kernel(
        _deg_body,
        out_type=jax.ShapeDtypeStruct((NC, N_PAD, FH), jnp.float32),
        mesh=mesh,
        scratch_types=[
            pltpu.VMEM((CHUNKS_PER_TILE, CHUNK), jnp.int32),
            pltpu.VMEM((CHUNK, FH), jnp.float32),
            pltpu.VMEM_SHARED((N_PAD, FH), jnp.float32),
            pltpu.SemaphoreType.DMA,
            pltpu.SemaphoreType.DMA,
            pltpu.SemaphoreType.DMA,
            pltpu.SemaphoreType.DMA,
        ],
    )


def _scatter_body(u_hbm, src_hbm, dst_hbm, zeros_hbm, out_hbm,
                  idx_s, idx_d, r0, r1, r2, r3, acc,
                  g0, g1, g2, g3, s0, s1, s2, s3):
    cid = lax.axis_index("c")
    sid = lax.axis_index("s")
    wid = cid * NS + sid
    rows = (r0, r1, r2, r3)
    gs = (g0, g1, g2, g3)
    ss = (s0, s1, s2, s3)
    pltpu.sync_copy(zeros_hbm, acc.at[pl.ds(sid * ROWS_PER_TILE, ROWS_PER_TILE)])
    plsc.subcore_barrier()

    for ph in range(NPHASE):
        base = wid * CPT + ph * PHASE_CHUNKS
        pltpu.sync_copy(src_hbm.at[pl.ds(base, PHASE_CHUNKS)], idx_s)
        pltpu.sync_copy(dst_hbm.at[pl.ds(base, PHASE_CHUNKS)], idx_d)
        pltpu.async_copy(u_hbm.at[idx_s.at[0]], rows[0], gs[0])
        pltpu.async_copy(u_hbm.at[idx_s.at[1]], rows[1], gs[1])
        for j in (0, 1):
            b = j
            pltpu.make_async_copy(u_hbm.at[idx_s.at[j]], rows[b], gs[b]).wait()
            pltpu.async_copy(rows[b], acc.at[idx_d.at[j]], ss[b], add=True)
            pltpu.async_copy(u_hbm.at[idx_s.at[j + 2]], rows[j + 2], gs[j + 2])

        def body(jj, carry):
            for bo in range(4):
                j = 2 + 4 * jj + bo
                b = (2 + bo) % 4
                bf = (b + 2) % 4
                pltpu.make_async_copy(u_hbm.at[idx_s.at[j]], rows[b], gs[b]).wait()
                pltpu.async_copy(rows[b], acc.at[idx_d.at[j]], ss[b], add=True)
                pltpu.make_async_copy(rows[bf], acc.at[idx_d.at[j - 2]], ss[bf]).wait()
                pltpu.async_copy(u_hbm.at[idx_s.at[j + 2]], rows[bf], gs[bf])
            return carry

        lax.fori_loop(0, (PHASE_CHUNKS - 4) // 4, body, 0)
        for j in (PHASE_CHUNKS - 2, PHASE_CHUNKS - 1):
            b = j % 4
            pltpu.make_async_copy(u_hbm.at[idx_s.at[j]], rows[b], gs[b]).wait()
            pltpu.async_copy(rows[b], acc.at[idx_d.at[j]], ss[b], add=True)
        for j in (PHASE_CHUNKS - 4, PHASE_CHUNKS - 3, PHASE_CHUNKS - 2, PHASE_CHUNKS - 1):
            b = j % 4
            pltpu.make_async_copy(rows[b], acc.at[idx_d.at[j]], ss[b]).wait()

    plsc.subcore_barrier()
    pltpu.sync_copy(acc.at[pl.ds(sid * ROWS_PER_TILE, ROWS_PER_TILE)],
                    out_hbm.at[cid, pl.ds(sid * ROWS_PER_TILE, ROWS_PER_TILE)])


@functools.cache
def _get_scatter_kernel():
    mesh = plsc.VectorSubcoreMesh(core_axis_name="c", subcore_axis_name="s")
    return pl.kernel(
        _scatter_body,
        out_type=jax.ShapeDtypeStruct((NC, N_PAD, FH), jnp.float32),
        mesh=mesh,
        scratch_types=[
            pltpu.VMEM((PHASE_CHUNKS, CH), jnp.int32),
            pltpu.VMEM((PHASE_CHUNKS, CH), jnp.int32),
            pltpu.VMEM((CH, FH), jnp.float32),
            pltpu.VMEM((CH, FH), jnp.float32),
            pltpu.VMEM((CH, FH), jnp.float32),
            pltpu.VMEM((CH, FH), jnp.float32),
            pltpu.VMEM_SHARED((N_PAD, FH), jnp.float32),
        ] + [pltpu.SemaphoreType.DMA] * 8,
    )


def _sc_degree(src2):
    ones = jnp.ones((CHUNK, FH), jnp.float32)
    zeros = jnp.zeros((ROWS_PER_TILE, FH), jnp.float32)
    return _get_deg_kernel()(src2, ones, zeros)


def _sc_scatter(u, src2, dst2):
    zeros = jnp.zeros((ROWS_PER_TILE, FH), jnp.float32)
    return _get_scatter_kernel()(u, src2, dst2, zeros)


_RB = 1000
_GRID = N // _RB

def _row_spec(w):
    return pl.BlockSpec((_RB, w), lambda i: (i, 0))


def _p0_body(dega, degb, x, dis_o, u0_o):
    deg = dega[...] + degb[...]
    dis = jnp.where(deg > 0, lax.rsqrt(jnp.maximum(deg, 1e-12)), 0.0)
    dis_o[...] = dis
    u0_o[...] = dis * x[...]


def _tc_p0(dega, degb, x):
    return pl.pallas_call(
        _p0_body,
        grid=(_GRID,),
        in_specs=[_row_spec(1), _row_spec(1), _row_spec(FH)],
        out_specs=[_row_spec(1), _row_spec(FH)],
        out_shape=[jax.ShapeDtypeStruct((N, 1), jnp.float32),
                   jax.ShapeDtypeStruct((N, FH), jnp.float32)],
    )(dega, degb, x)


def _p1_body(a0, a1, dis, tx1_o, u1_o):
    dis_ = dis[...]
    tx1 = -dis_ * (a0[...] + a1[...])
    tx1_o[...] = tx1
    u1_o[...] = dis_ * tx1


def _tc_p1(a0, a1, dis):
    return pl.pallas_call(
        _p1_body,
        grid=(_GRID,),
        in_specs=[_row_spec(FH), _row_spec(FH), _row_spec(1)],
        out_specs=[_row_spec(FH), _row_spec(FH)],
        out_shape=[jax.ShapeDtypeStruct((N, FH), jnp.float32),
                   jax.ShapeDtypeStruct((N, FH), jnp.float32)],
    )(a0, a1, dis)


def _p2_body(a0, a1, dis, x, tx2_o, u2_o):
    dis_ = dis[...]
    tx2 = -2.0 * dis_ * (a0[...] + a1[...]) - x[...]
    tx2_o[...] = tx2
    u2_o[...] = dis_ * tx2


def _tc_p2(a0, a1, dis, x):
    return pl.pallas_call(
        _p2_body,
        grid=(_GRID,),
        in_specs=[_row_spec(FH), _row_spec(FH), _row_spec(1), _row_spec(FH)],
        out_specs=[_row_spec(FH), _row_spec(FH)],
        out_shape=[jax.ShapeDtypeStruct((N, FH), jnp.float32),
                   jax.ShapeDtypeStruct((N, FH), jnp.float32)],
    )(a0, a1, dis, x)


def _head_body(a0, a1, dis, x, tx1, tx2, wc, bias, wco, wlin, blin, y_o):
    dis_ = dis[...]
    tx3 = -2.0 * dis_ * (a0[...] + a1[...]) - tx1[...]
    z = jnp.dot(x[...], wc[0], preferred_element_type=jnp.float32)
    z += jnp.dot(tx1[...], wc[1], preferred_element_type=jnp.float32)
    z += jnp.dot(tx2[...], wc[2], preferred_element_type=jnp.float32)
    z += jnp.dot(tx3, wc[3], preferred_element_type=jnp.float32)
    z += bias[...]
    gi = jax.nn.sigmoid(z[:, :FH])
    gt = jnp.tanh(z[:, 2 * FH:3 * FH])
    c = gi * gt
    go = jax.nn.sigmoid(z[:, 3 * FH:] + wco[...] * c)
    h = go * jnp.tanh(c)
    y_o[...] = jnp.dot(h, wlin[...], preferred_element_type=jnp.float32) + blin[...]


def _tc_head(a0, a1, dis, x, tx1, tx2, wc, bias, wco, wlin, blin):
    full = lambda s: pl.BlockSpec(s, lambda i: tuple(0 for _ in s))
    return pl.pallas_call(
        _head_body,
        grid=(_GRID,),
        in_specs=[_row_spec(FH), _row_spec(FH), _row_spec(1), _row_spec(FH),
                  _row_spec(FH), _row_spec(FH),
                  full((4, FH, 4 * FH)), full((1, 4 * FH)), full((1, FH)),
                  full((FH, 1)), full((1, 1))],
        out_specs=_row_spec(1),
        out_shape=jax.ShapeDtypeStruct((N, 1), jnp.float32),
    )(a0, a1, dis, x, tx1, tx2, wc, bias, wco, wlin, blin)


def kernel(x, edge_index, params):
    p = params
    src2 = edge_index[0].reshape(NW * CHUNKS_PER_TILE, CHUNK)
    npad = E_PAD - E
    pad_src = jnp.arange(npad, dtype=jnp.int32) % N
    pad_dst = N + (jnp.arange(npad, dtype=jnp.int32) % (N_PAD - N))
    src_g = jnp.concatenate([edge_index[0], pad_src]).reshape(NW * CPT, CH)
    dst_g = jnp.concatenate([edge_index[1], pad_dst]).reshape(NW * CPT, CH)

    deg2 = _sc_degree(src2)
    dega = deg2[0, :N, 0:1]
    degb = deg2[1, :N, 0:1]
    dis, u0 = _tc_p0(dega, degb, x)

    a1 = _sc_scatter(u0, src_g, dst_g)
    tx1, u1 = _tc_p1(a1[0, :N], a1[1, :N], dis)
    a2 = _sc_scatter(u1, src_g, dst_g)
    tx2, u2 = _tc_p2(a2[0, :N], a2[1, :N], dis, x)
    a3 = _sc_scatter(u2, src_g, dst_g)

    wc = jnp.stack([
        jnp.concatenate([p["W_x_" + g][k] for g in ("i", "f", "c", "o")], axis=1)
        for k in range(4)])
    bias = jnp.concatenate(
        [(p["b_x_" + g] + p["b_h_" + g] + p["b_" + g][0])[None, :]
         for g in ("i", "f", "c", "o")], axis=1)
    blin = p["b_lin"].reshape(1, 1)

    return _tc_head(a3[0, :N], a3[1, :N], dis, x, tx1, tx2,
                    wc, bias, p["w_c_o"], p["W_lin"], blin)


# speedup vs baseline: 1.1150x; 1.1150x over previous
import functools

import jax
import jax.numpy as jnp
from jax import lax
from jax.experimental import pallas as pl
from jax.experimental.pallas import tpu as pltpu
from jax.experimental.pallas import tpu_sc as plsc

N = 10000
E = 320000
FH = 128
NC = 2
NS = 16
NW = NC * NS
CHUNK = 125
EDGES_PER_TILE = E // NW
CHUNKS_PER_TILE = EDGES_PER_TILE // CHUNK
N_PAD = 10240
ROWS_PER_TILE = N_PAD // NS
NPHASE = 2
PHASE_CHUNKS = CHUNKS_PER_TILE // NPHASE
DEGW = 16

def _deg_body(src_hbm, ones_hbm, zeros_hbm, out_hbm, idx_s, ones_v, acc,
              d0, d1, d2, d3):
    cid = lax.axis_index("c")
    sid = lax.axis_index("s")
    wid = cid * NS + sid
    dsems = (d0, d1, d2, d3)
    pltpu.sync_copy(zeros_hbm, acc.at[pl.ds(sid * ROWS_PER_TILE, ROWS_PER_TILE)])
    pltpu.sync_copy(ones_hbm, ones_v)
    pltpu.sync_copy(src_hbm.at[pl.ds(wid * CHUNKS_PER_TILE, CHUNKS_PER_TILE)], idx_s)
    plsc.subcore_barrier()

    def body(g, carry):
        for k in range(4):
            pltpu.async_copy(ones_v, acc.at[idx_s.at[4 * g + k]], dsems[k], add=True)
        for k in range(4):
            pltpu.make_async_copy(ones_v, acc.at[idx_s.at[4 * g + k]], dsems[k]).wait()
        return carry

    lax.fori_loop(0, CHUNKS_PER_TILE // 4, body, 0)
    plsc.subcore_barrier()
    pltpu.sync_copy(acc.at[pl.ds(sid * ROWS_PER_TILE, ROWS_PER_TILE)],
                    out_hbm.at[cid, pl.ds(sid * ROWS_PER_TILE, ROWS_PER_TILE)])


@functools.cache
def _get_deg_kernel():
    mesh = plsc.VectorSubcoreMesh(core_axis_name="c", subcore_axis_name="s")
    return pl.kernel(
        _deg_body,
        out_type=jax.ShapeDtypeStruct((NC, N_PAD, FH), jnp.float32),
        mesh=mesh,
        scratch_types=[
            pltpu.VMEM((CHUNKS_PER_TILE, CHUNK), jnp.int32),
            pltpu.VMEM((CHUNK, FH), jnp.float32),
            pltpu.VMEM_SHARED((N_PAD, FH), jnp.float32),
            pltpu.SemaphoreType.DMA,
            pltpu.SemaphoreType.DMA,
            pltpu.SemaphoreType.DMA,
            pltpu.SemaphoreType.DMA,
        ],
    )


def _scatter_body(u_hbm, src_hbm, dst_hbm, zeros_hbm, out_hbm,
                  idx_s, idx_d, rows0, rows1, acc, sem0, sem1):
    cid = lax.axis_index("c")
    sid = lax.axis_index("s")
    wid = cid * NS + sid
    pltpu.sync_copy(zeros_hbm, acc.at[pl.ds(sid * ROWS_PER_TILE, ROWS_PER_TILE)])
    plsc.subcore_barrier()

    rows = (rows0, rows1)
    sems = (sem0, sem1)
    for ph in range(NPHASE):
        base = wid * CHUNKS_PER_TILE + ph * PHASE_CHUNKS
        pltpu.sync_copy(src_hbm.at[pl.ds(base, PHASE_CHUNKS)], idx_s)
        pltpu.sync_copy(dst_hbm.at[pl.ds(base, PHASE_CHUNKS)], idx_d)
        pltpu.async_copy(u_hbm.at[idx_s.at[0]], rows0, sem0)
        pltpu.async_copy(u_hbm.at[idx_s.at[1]], rows1, sem1)

        def body(jj, carry):
            for b in range(2):
                j = 2 * jj + b
                pltpu.make_async_copy(u_hbm.at[idx_s.at[j]], rows[b], sems[b]).wait()
                pltpu.sync_copy(rows[b], acc.at[idx_d.at[j]], add=True)
                pltpu.async_copy(u_hbm.at[idx_s.at[j + 2]], rows[b], sems[b])
            return carry

        lax.fori_loop(0, PHASE_CHUNKS // 2 - 1, body, 0)
        for b in range(2):
            j = PHASE_CHUNKS - 2 + b
            pltpu.make_async_copy(u_hbm.at[idx_s.at[j]], rows[b], sems[b]).wait()
            pltpu.sync_copy(rows[b], acc.at[idx_d.at[j]], add=True)

    plsc.subcore_barrier()
    pltpu.sync_copy(acc.at[pl.ds(sid * ROWS_PER_TILE, ROWS_PER_TILE)],
                    out_hbm.at[cid, pl.ds(sid * ROWS_PER_TILE, ROWS_PER_TILE)])


@functools.cache
def _get_scatter_kernel():
    mesh = plsc.VectorSubcoreMesh(core_axis_name="c", subcore_axis_name="s")
    return pl.kernel(
        _scatter_body,
        out_type=jax.ShapeDtypeStruct((NC, N_PAD, FH), jnp.float32),
        mesh=mesh,
        scratch_types=[
            pltpu.VMEM((PHASE_CHUNKS, CHUNK), jnp.int32),
            pltpu.VMEM((PHASE_CHUNKS, CHUNK), jnp.int32),
            pltpu.VMEM((CHUNK, FH), jnp.float32),
            pltpu.VMEM((CHUNK, FH), jnp.float32),
            pltpu.VMEM_SHARED((N_PAD, FH), jnp.float32),
            pltpu.SemaphoreType.DMA,
            pltpu.SemaphoreType.DMA,
        ],
    )


def _sc_degree(src2):
    ones = jnp.ones((CHUNK, FH), jnp.float32)
    zeros = jnp.zeros((ROWS_PER_TILE, FH), jnp.float32)
    return _get_deg_kernel()(src2, ones, zeros)


def _sc_scatter(u, src2, dst2):
    zeros = jnp.zeros((ROWS_PER_TILE, FH), jnp.float32)
    return _get_scatter_kernel()(u, src2, dst2, zeros)


_RB = 1000
_GRID = N // _RB

def _row_spec(w):
    return pl.BlockSpec((_RB, w), lambda i: (i, 0))


def _p0_body(dega, degb, x, dis_o, u0_o):
    deg = dega[...] + degb[...]
    dis = jnp.where(deg > 0, lax.rsqrt(jnp.maximum(deg, 1e-12)), 0.0)
    dis_o[...] = dis
    u0_o[...] = dis * x[...]


def _tc_p0(dega, degb, x):
    return pl.pallas_call(
        _p0_body,
        grid=(_GRID,),
        in_specs=[_row_spec(1), _row_spec(1), _row_spec(FH)],
        out_specs=[_row_spec(1), _row_spec(FH)],
        out_shape=[jax.ShapeDtypeStruct((N, 1), jnp.float32),
                   jax.ShapeDtypeStruct((N, FH), jnp.float32)],
    )(dega, degb, x)


def _p1_body(a0, a1, dis, tx1_o, u1_o):
    dis_ = dis[...]
    tx1 = -dis_ * (a0[...] + a1[...])
    tx1_o[...] = tx1
    u1_o[...] = dis_ * tx1


def _tc_p1(a0, a1, dis):
    return pl.pallas_call(
        _p1_body,
        grid=(_GRID,),
        in_specs=[_row_spec(FH), _row_spec(FH), _row_spec(1)],
        out_specs=[_row_spec(FH), _row_spec(FH)],
        out_shape=[jax.ShapeDtypeStruct((N, FH), jnp.float32),
                   jax.ShapeDtypeStruct((N, FH), jnp.float32)],
    )(a0, a1, dis)


def _p2_body(a0, a1, dis, x, tx2_o, u2_o):
    dis_ = dis[...]
    tx2 = -2.0 * dis_ * (a0[...] + a1[...]) - x[...]
    tx2_o[...] = tx2
    u2_o[...] = dis_ * tx2


def _tc_p2(a0, a1, dis, x):
    return pl.pallas_call(
        _p2_body,
        grid=(_GRID,),
        in_specs=[_row_spec(FH), _row_spec(FH), _row_spec(1), _row_spec(FH)],
        out_specs=[_row_spec(FH), _row_spec(FH)],
        out_shape=[jax.ShapeDtypeStruct((N, FH), jnp.float32),
                   jax.ShapeDtypeStruct((N, FH), jnp.float32)],
    )(a0, a1, dis, x)


def _head_body(a0, a1, dis, x, tx1, tx2, wc, bias, wco, wlin, blin, y_o):
    dis_ = dis[...]
    tx3 = -2.0 * dis_ * (a0[...] + a1[...]) - tx1[...]
    z = jnp.dot(x[...], wc[0], preferred_element_type=jnp.float32)
    z += jnp.dot(tx1[...], wc[1], preferred_element_type=jnp.float32)
    z += jnp.dot(tx2[...], wc[2], preferred_element_type=jnp.float32)
    z += jnp.dot(tx3, wc[3], preferred_element_type=jnp.float32)
    z += bias[...]
    gi = jax.nn.sigmoid(z[:, :FH])
    gt = jnp.tanh(z[:, 2 * FH:3 * FH])
    c = gi * gt
    go = jax.nn.sigmoid(z[:, 3 * FH:] + wco[...] * c)
    h = go * jnp.tanh(c)
    y_o[...] = jnp.dot(h, wlin[...], preferred_element_type=jnp.float32) + blin[...]


def _tc_head(a0, a1, dis, x, tx1, tx2, wc, bias, wco, wlin, blin):
    full = lambda s: pl.BlockSpec(s, lambda i: tuple(0 for _ in s))
    return pl.pallas_call(
        _head_body,
        grid=(_GRID,),
        in_specs=[_row_spec(FH), _row_spec(FH), _row_spec(1), _row_spec(FH),
                  _row_spec(FH), _row_spec(FH),
                  full((4, FH, 4 * FH)), full((1, 4 * FH)), full((1, FH)),
                  full((FH, 1)), full((1, 1))],
        out_specs=_row_spec(1),
        out_shape=jax.ShapeDtypeStruct((N, 1), jnp.float32),
    )(a0, a1, dis, x, tx1, tx2, wc, bias, wco, wlin, blin)


def kernel(x, edge_index, params):
    p = params
    src2 = edge_index[0].reshape(NW * CHUNKS_PER_TILE, CHUNK)
    dst2 = edge_index[1].reshape(NW * CHUNKS_PER_TILE, CHUNK)

    deg2 = _sc_degree(src2)
    dega = deg2[0, :N, 0:1]
    degb = deg2[1, :N, 0:1]
    dis, u0 = _tc_p0(dega, degb, x)

    a1 = _sc_scatter(u0, src2, dst2)
    tx1, u1 = _tc_p1(a1[0, :N], a1[1, :N], dis)
    a2 = _sc_scatter(u1, src2, dst2)
    tx2, u2 = _tc_p2(a2[0, :N], a2[1, :N], dis, x)
    a3 = _sc_scatter(u2, src2, dst2)

    wc = jnp.stack([
        jnp.concatenate([p["W_x_" + g][k] for g in ("i", "f", "c", "o")], axis=1)
        for k in range(4)])
    bias = jnp.concatenate(
        [(p["b_x_" + g] + p["b_h_" + g] + p["b_" + g][0])[None, :]
         for g in ("i", "f", "c", "o")], axis=1)
    blin = p["b_lin"].reshape(1, 1)

    return _tc_head(a3[0, :N], a3[1, :N], dis, x, tx1, tx2,
                    wc, bias, p["w_c_o"], p["W_lin"], blin)
